# baseline (speedup 1.0000x reference)
import functools

import jax
import jax.numpy as jnp
from jax import lax
from jax.experimental import pallas as pl
from jax.experimental.pallas import tpu as pltpu
from jax.experimental.pallas import tpu_sc as plsc

NF = 256
NS = 8192
B = 4096
TEMP = 0.05
EPS = 0.1

BB = 256
BK = 8192

NC = 2
NSUB = 16
NW = NC * NSUB
RPW = B // NW
LANES = 16


def _row_normalize_scaled(x, scale):
    n = jnp.sqrt(jnp.sum(x * x, axis=1, keepdims=True))
    return x * (scale / jnp.maximum(n, 1e-12))


def _tc_kernel(x1_ref, x2_ref, f_ref, lse_ref, soft_ref):
    x1 = _row_normalize_scaled(x1_ref[...], 1.0 / TEMP).astype(jnp.bfloat16)
    x2 = _row_normalize_scaled(x2_ref[...], 1.0 / TEMP).astype(jnp.bfloat16)

    init = (
        jnp.zeros((BB, 1), jnp.float32),
        jnp.zeros((BB, 1), jnp.float32),
        jnp.zeros((BB, 1), jnp.float32),
    )

    def body(k, carry):
        s1, s2, w = carry
        f = f_ref[pl.ds(k * BK, BK), :].astype(jnp.bfloat16)
        out = jax.lax.dot_general(
            x1, f, (((1,), (1,)), ((), ())),
            preferred_element_type=jnp.float32)
        reg = jax.lax.dot_general(
            x2, f, (((1,), (1,)), ((), ())),
            preferred_element_type=jnp.float32)

        s1 = s1 + jnp.sum(jnp.exp(out), axis=1, keepdims=True)
        e2 = jnp.exp(reg)
        s2 = s2 + jnp.sum(e2, axis=1, keepdims=True)
        w = w + jnp.sum(e2 * out, axis=1, keepdims=True)
        return s1, s2, w

    s1, s2, w = jax.lax.fori_loop(0, NS // BK, body, init)

    lse_ref[0, :, :] = jnp.broadcast_to(jnp.sum(jnp.log(s1)), (1, 128))
    soft_ref[0, :, :] = jnp.broadcast_to(jnp.sum(w / s2), (1, 128))


@jax.jit
def _run(x1, x2, t, f):
    nb = B // BB
    dot_raw, sumsq = _sc_target_dot(x1, t, f)
    lse_sums, soft_sums = pl.pallas_call(
        _tc_kernel,
        grid=(nb,),
        in_specs=[
            pl.BlockSpec((BB, NF), lambda i: (i, 0)),
            pl.BlockSpec((BB, NF), lambda i: (i, 0)),
            pl.BlockSpec((NS, NF), lambda i: (0, 0)),
        ],
        out_specs=[
            pl.BlockSpec((1, 1, 128), lambda i: (i, 0, 0)),
            pl.BlockSpec((1, 1, 128), lambda i: (i, 0, 0)),
        ],
        out_shape=[
            jax.ShapeDtypeStruct((nb, 1, 128), jnp.float32),
            jax.ShapeDtypeStruct((nb, 1, 128), jnp.float32),
        ],
    )(x1, x2, f)

    ot = dot_raw / (jnp.maximum(jnp.sqrt(sumsq), 1e-12) * TEMP)
    sum_ot = jnp.sum(ot)
    sum_lse = jnp.sum(lse_sums[:, 0, 0])
    sum_soft = jnp.sum(soft_sums[:, 0, 0])
    loss_c = (sum_lse - sum_ot) / B
    loss_s = (sum_lse - (1.0 - EPS) * sum_soft - EPS * sum_ot) / B
    return loss_c, loss_s


@functools.partial(
    pl.kernel,
    mesh=plsc.VectorSubcoreMesh(core_axis_name="c", subcore_axis_name="s"),
    compiler_params=pltpu.CompilerParams(needs_layout_passes=False),
    out_type=[
        jax.ShapeDtypeStruct((B,), jnp.float32),
        jax.ShapeDtypeStruct((B,), jnp.float32),
    ],
    scratch_types=[
        pltpu.VMEM((RPW,), jnp.int32),
        pltpu.VMEM((RPW, NF), jnp.float32),
        pltpu.VMEM((RPW, NF), jnp.float32),
        pltpu.VMEM((RPW,), jnp.float32),
        pltpu.VMEM((RPW,), jnp.float32),
        pltpu.SemaphoreType.DMA,
    ],
)
def _sc_target_dot(x1_hbm, t_hbm, f_hbm, dot_hbm, ss_hbm,
                   idx_v, rows_v, x_v, dot_v, ss_v, sem):
    wid = lax.axis_index("s") * NC + lax.axis_index("c")
    base = wid * RPW
    pltpu.sync_copy(t_hbm.at[pl.ds(base, RPW)], idx_v)
    gather = pltpu.async_copy(f_hbm.at[idx_v], rows_v, sem)
    pltpu.sync_copy(x1_hbm.at[pl.ds(base, RPW)], x_v)
    gather.wait()

    def group_body(g, _):
        rows = g * LANES + lax.iota(jnp.int32, LANES)
        diag = lax.iota(jnp.int32, LANES)
        accd = jnp.zeros((LANES,), jnp.float32)
        accs = jnp.zeros((LANES,), jnp.float32)
        for d in range(NF):
            colv = (diag + d) & (NF - 1)
            xv = plsc.load_gather(x_v, [rows, colv])
            gv = plsc.load_gather(rows_v, [rows, colv])
            accd = accd + xv * gv
            accs = accs + xv * xv
        dot_v[pl.ds(g * LANES, LANES)] = accd
        ss_v[pl.ds(g * LANES, LANES)] = accs
        return 0

    lax.fori_loop(0, RPW // LANES, group_body, 0)

    pltpu.sync_copy(dot_v, dot_hbm.at[pl.ds(base, RPW)])
    pltpu.sync_copy(ss_v, ss_hbm.at[pl.ds(base, RPW)])


def kernel(inputs1, inputs2, targets, features):
    return _run(inputs1, inputs2, targets.astype(jnp.int32), features)


# speedup vs baseline: 1.0875x; 1.0875x over previous
import functools

import jax
import jax.numpy as jnp
from jax import lax
from jax.experimental import pallas as pl
from jax.experimental.pallas import tpu as pltpu
from jax.experimental.pallas import tpu_sc as plsc

NF = 256
NS = 8192
B = 4096
TEMP = 0.05
EPS = 0.1

BB = 1024
BK = 8192

NC = 2
NSUB = 16
NW = NC * NSUB
RPW = B // NW
LANES = 16


def _row_normalize_scaled(x, scale):
    n = jnp.sqrt(jnp.sum(x * x, axis=1, keepdims=True))
    return x * (scale / jnp.maximum(n, 1e-12))


def _tc_kernel(x1_ref, x2_ref, f_ref, lse_ref, soft_ref):
    x1 = _row_normalize_scaled(x1_ref[...], 1.0 / TEMP).astype(jnp.bfloat16)
    x2 = _row_normalize_scaled(x2_ref[...], 1.0 / TEMP).astype(jnp.bfloat16)

    init = (
        jnp.zeros((BB, 1), jnp.float32),
        jnp.zeros((BB, 1), jnp.float32),
        jnp.zeros((BB, 1), jnp.float32),
    )

    def body(k, carry):
        s1, s2, w = carry
        f = f_ref[pl.ds(k * BK, BK), :].astype(jnp.bfloat16)
        out = jax.lax.dot_general(
            x1, f, (((1,), (1,)), ((), ())),
            preferred_element_type=jnp.float32)
        reg = jax.lax.dot_general(
            x2, f, (((1,), (1,)), ((), ())),
            preferred_element_type=jnp.float32)

        s1 = s1 + jnp.sum(jnp.exp(out), axis=1, keepdims=True)
        e2 = jnp.exp(reg)
        s2 = s2 + jnp.sum(e2, axis=1, keepdims=True)
        w = w + jnp.sum(e2 * out, axis=1, keepdims=True)
        return s1, s2, w

    s1, s2, w = jax.lax.fori_loop(0, NS // BK, body, init)

    lse_ref[0, :, :] = jnp.broadcast_to(jnp.sum(jnp.log(s1)), (1, 128))
    soft_ref[0, :, :] = jnp.broadcast_to(jnp.sum(w / s2), (1, 128))


@jax.jit
def _run(x1, x2, t, f):
    nb = B // BB
    dot_raw, sumsq = _sc_target_dot(x1, t, f)
    lse_sums, soft_sums = pl.pallas_call(
        _tc_kernel,
        grid=(nb,),
        in_specs=[
            pl.BlockSpec((BB, NF), lambda i: (i, 0)),
            pl.BlockSpec((BB, NF), lambda i: (i, 0)),
            pl.BlockSpec((NS, NF), lambda i: (0, 0)),
        ],
        out_specs=[
            pl.BlockSpec((1, 1, 128), lambda i: (i, 0, 0)),
            pl.BlockSpec((1, 1, 128), lambda i: (i, 0, 0)),
        ],
        out_shape=[
            jax.ShapeDtypeStruct((nb, 1, 128), jnp.float32),
            jax.ShapeDtypeStruct((nb, 1, 128), jnp.float32),
        ],
    )(x1, x2, f)

    ot = dot_raw / (jnp.maximum(jnp.sqrt(sumsq), 1e-12) * TEMP)
    sum_ot = jnp.sum(ot)
    sum_lse = jnp.sum(lse_sums[:, 0, 0])
    sum_soft = jnp.sum(soft_sums[:, 0, 0])
    loss_c = (sum_lse - sum_ot) / B
    loss_s = (sum_lse - (1.0 - EPS) * sum_soft - EPS * sum_ot) / B
    return loss_c, loss_s


@functools.partial(
    pl.kernel,
    mesh=plsc.VectorSubcoreMesh(core_axis_name="c", subcore_axis_name="s"),
    compiler_params=pltpu.CompilerParams(needs_layout_passes=False),
    out_type=[
        jax.ShapeDtypeStruct((B,), jnp.float32),
        jax.ShapeDtypeStruct((B,), jnp.float32),
    ],
    scratch_types=[
        pltpu.VMEM((RPW,), jnp.int32),
        pltpu.VMEM((RPW, NF), jnp.float32),
        pltpu.VMEM((RPW, NF), jnp.float32),
        pltpu.VMEM((RPW,), jnp.float32),
        pltpu.VMEM((RPW,), jnp.float32),
        pltpu.SemaphoreType.DMA,
    ],
)
def _sc_target_dot(x1_hbm, t_hbm, f_hbm, dot_hbm, ss_hbm,
                   idx_v, rows_v, x_v, dot_v, ss_v, sem):
    wid = lax.axis_index("s") * NC + lax.axis_index("c")
    base = wid * RPW
    pltpu.sync_copy(t_hbm.at[pl.ds(base, RPW)], idx_v)
    gather = pltpu.async_copy(f_hbm.at[idx_v], rows_v, sem)
    pltpu.sync_copy(x1_hbm.at[pl.ds(base, RPW)], x_v)
    gather.wait()

    def group_body(g, _):
        rows = g * LANES + lax.iota(jnp.int32, LANES)
        diag = lax.iota(jnp.int32, LANES)
        accd = jnp.zeros((LANES,), jnp.float32)
        accs = jnp.zeros((LANES,), jnp.float32)
        for d in range(NF):
            colv = (diag + d) & (NF - 1)
            xv = plsc.load_gather(x_v, [rows, colv])
            gv = plsc.load_gather(rows_v, [rows, colv])
            accd = accd + xv * gv
            accs = accs + xv * xv
        dot_v[pl.ds(g * LANES, LANES)] = accd
        ss_v[pl.ds(g * LANES, LANES)] = accs
        return 0

    lax.fori_loop(0, RPW // LANES, group_body, 0)

    pltpu.sync_copy(dot_v, dot_hbm.at[pl.ds(base, RPW)])
    pltpu.sync_copy(ss_v, ss_hbm.at[pl.ds(base, RPW)])


def kernel(inputs1, inputs2, targets, features):
    return _run(inputs1, inputs2, targets.astype(jnp.int32), features)


# speedup vs baseline: 1.0916x; 1.0038x over previous
import jax
import jax.numpy as jnp
from jax.experimental import pallas as pl

NF = 256
NS = 8192
B = 4096
TEMP = 0.05
EPS = 0.1

BB = 512
BK = 8192


def _row_normalize_scaled(x, scale):
    n = jnp.sqrt(jnp.sum(x * x, axis=1, keepdims=True))
    return x * (scale / jnp.maximum(n, 1e-12))


def _tc_kernel(x1_ref, x2_ref, t_ref, f_ref, outc_ref, outs_ref):
    x1 = _row_normalize_scaled(x1_ref[...], 1.0 / TEMP).astype(jnp.bfloat16)
    x2 = _row_normalize_scaled(x2_ref[...], 1.0 / TEMP).astype(jnp.bfloat16)
    t = t_ref[0, 0, :]

    init = (
        jnp.zeros((BB, 1), jnp.float32),
        jnp.zeros((BB, 1), jnp.float32),
        jnp.zeros((BB, 1), jnp.float32),
        jnp.zeros((BB, 1), jnp.float32),
    )

    def body(k, carry):
        s1, s2, w, ot = carry
        f = f_ref[pl.ds(k * BK, BK), :].astype(jnp.bfloat16)
        out = jax.lax.dot_general(
            x1, f, (((1,), (1,)), ((), ())),
            preferred_element_type=jnp.float32)
        reg = jax.lax.dot_general(
            x2, f, (((1,), (1,)), ((), ())),
            preferred_element_type=jnp.float32)

        s1 = s1 + jnp.sum(jnp.exp(out), axis=1, keepdims=True)
        e2 = jnp.exp(reg)
        s2 = s2 + jnp.sum(e2, axis=1, keepdims=True)
        w = w + jnp.sum(e2 * out, axis=1, keepdims=True)
        cols = k * BK + jax.lax.broadcasted_iota(jnp.int32, (BB, BK), 1)
        ot = ot + jnp.sum(
            jnp.where(cols == t[:, None], out, 0.0), axis=1, keepdims=True)
        return s1, s2, w, ot

    s1, s2, w, ot = jax.lax.fori_loop(0, NS // BK, body, init)

    lse1 = jnp.log(s1)
    loss_c = lse1 - ot
    loss_s = lse1 - (1.0 - EPS) * (w / s2) - EPS * ot
    outc_ref[0, :, :] = jnp.broadcast_to(jnp.sum(loss_c), (1, 128))
    outs_ref[0, :, :] = jnp.broadcast_to(jnp.sum(loss_s), (1, 128))


@jax.jit
def _run(x1, x2, t3, f):
    nb = B // BB
    outc, outs = pl.pallas_call(
        _tc_kernel,
        grid=(nb,),
        in_specs=[
            pl.BlockSpec((BB, NF), lambda i: (i, 0)),
            pl.BlockSpec((BB, NF), lambda i: (i, 0)),
            pl.BlockSpec((1, 1, BB), lambda i: (i, 0, 0)),
            pl.BlockSpec((NS, NF), lambda i: (0, 0)),
        ],
        out_specs=[
            pl.BlockSpec((1, 1, 128), lambda i: (i, 0, 0)),
            pl.BlockSpec((1, 1, 128), lambda i: (i, 0, 0)),
        ],
        out_shape=[
            jax.ShapeDtypeStruct((nb, 1, 128), jnp.float32),
            jax.ShapeDtypeStruct((nb, 1, 128), jnp.float32),
        ],
    )(x1, x2, t3, f)
    return jnp.sum(outc[:, 0, 0]) / B, jnp.sum(outs[:, 0, 0]) / B


def kernel(inputs1, inputs2, targets, features):
    t3 = targets.astype(jnp.int32).reshape(B // BB, 1, BB)
    return _run(inputs1, inputs2, t3, features)


# speedup vs baseline: 1.1037x; 1.0111x over previous
import jax
import jax.numpy as jnp
from jax.experimental import pallas as pl

NF = 256
NS = 8192
B = 4096
TEMP = 0.05
EPS = 0.1

BB = 1024
BK = 8192


def _row_normalize_scaled(x, scale):
    n = jnp.sqrt(jnp.sum(x * x, axis=1, keepdims=True))
    return x * (scale / jnp.maximum(n, 1e-12))


def _tc_kernel(x1_ref, x2_ref, t_ref, f_ref, outc_ref, outs_ref):
    x1 = _row_normalize_scaled(x1_ref[...], 1.0 / TEMP).astype(jnp.bfloat16)
    x2 = _row_normalize_scaled(x2_ref[...], 1.0 / TEMP).astype(jnp.bfloat16)
    t = t_ref[0, 0, :]

    init = (
        jnp.zeros((BB, 1), jnp.float32),
        jnp.zeros((BB, 1), jnp.float32),
        jnp.zeros((BB, 1), jnp.float32),
        jnp.zeros((BB, 1), jnp.float32),
    )

    def body(k, carry):
        s1, s2, w, ot = carry
        f = f_ref[pl.ds(k * BK, BK), :].astype(jnp.bfloat16)
        out = jax.lax.dot_general(
            x1, f, (((1,), (1,)), ((), ())),
            preferred_element_type=jnp.float32)
        reg = jax.lax.dot_general(
            x2, f, (((1,), (1,)), ((), ())),
            preferred_element_type=jnp.float32)

        s1 = s1 + jnp.sum(jnp.exp(out), axis=1, keepdims=True)
        e2 = jnp.exp(reg)
        s2 = s2 + jnp.sum(e2, axis=1, keepdims=True)
        w = w + jnp.sum(e2 * out, axis=1, keepdims=True)
        cols = k * BK + jax.lax.broadcasted_iota(jnp.int32, (BB, BK), 1)
        ot = ot + jnp.sum(
            jnp.where(cols == t[:, None], out, 0.0), axis=1, keepdims=True)
        return s1, s2, w, ot

    s1, s2, w, ot = jax.lax.fori_loop(0, NS // BK, body, init)

    lse1 = jnp.log(s1)
    loss_c = lse1 - ot
    loss_s = lse1 - (1.0 - EPS) * (w / s2) - EPS * ot
    outc_ref[0, :, :] = jnp.broadcast_to(jnp.sum(loss_c), (1, 128))
    outs_ref[0, :, :] = jnp.broadcast_to(jnp.sum(loss_s), (1, 128))


@jax.jit
def _run(x1, x2, t3, f):
    nb = B // BB
    outc, outs = pl.pallas_call(
        _tc_kernel,
        grid=(nb,),
        in_specs=[
            pl.BlockSpec((BB, NF), lambda i: (i, 0)),
            pl.BlockSpec((BB, NF), lambda i: (i, 0)),
            pl.BlockSpec((1, 1, BB), lambda i: (i, 0, 0)),
            pl.BlockSpec((NS, NF), lambda i: (0, 0)),
        ],
        out_specs=[
            pl.BlockSpec((1, 1, 128), lambda i: (i, 0, 0)),
            pl.BlockSpec((1, 1, 128), lambda i: (i, 0, 0)),
        ],
        out_shape=[
            jax.ShapeDtypeStruct((nb, 1, 128), jnp.float32),
            jax.ShapeDtypeStruct((nb, 1, 128), jnp.float32),
        ],
    )(x1, x2, t3, f)
    return jnp.sum(outc[:, 0, 0]) / B, jnp.sum(outs[:, 0, 0]) / B


def kernel(inputs1, inputs2, targets, features):
    t3 = targets.astype(jnp.int32).reshape(B // BB, 1, BB)
    return _run(inputs1, inputs2, t3, features)
